# Initial kernel scaffold; baseline (speedup 1.0000x reference)
import jax
import jax.numpy as jnp
from jax.experimental import pallas as pl


def kernel(input, c1_Wl, c1_bl, c1_Wr, p1_Wrel, p1_brel, p1_Wroot, c2_Wl, c2_bl, c2_Wr, p2_Wrel, p2_brel, p2_Wroot, fc1_W, fc1_b, bn1_g, bn1_b, fc2_W, fc2_b, bn2_g, bn2_b, fc3_W, fc3_b):
    raise NotImplementedError("write your pallas kernel here")


# speedup vs baseline: 2660.5861x; 2660.5861x over previous
import numpy as np
import jax
import jax.numpy as jnp
from jax.experimental import pallas as pl
from jax.experimental.pallas import tpu as pltpu

_B, _N, _INC, _HID, _OUT = 64, 512, 195, 128, 64
_K1, _K2 = 256, 128


def _dot_t(a, b, precision=None):
    return jax.lax.dot_general(a, b, (((1,), (1,)), ((), ())),
                               precision=precision,
                               preferred_element_type=jnp.float32)


def _dot_m(a, b, precision=None):
    return jax.lax.dot_general(a, b, (((0,), (0,)), ((), ())),
                               precision=precision,
                               preferred_element_type=jnp.float32)


def _graph_body(x_ref, c1_Wl_ref, c1_bl_ref, c1_Wr_ref,
                p1_Wrel_ref, p1_brel_ref, p1_Wroot_ref,
                c2_Wl_ref, c2_bl_ref, c2_Wr_ref,
                p2_Wrel_ref, p2_brel_ref, p2_Wroot_ref,
                g_ref):
    f32 = jnp.float32
    x = x_ref[0]

    row_i = jax.lax.broadcasted_iota(jnp.int32, (_N, _N), 0)
    col_i = jax.lax.broadcasted_iota(jnp.int32, (_N, _N), 1)
    eye_b = row_i == col_i
    eye_f = eye_b.astype(f32)

    xm = x - jnp.mean(x, axis=1, keepdims=True)
    cov = _dot_t(xm, xm, precision=jax.lax.Precision.HIGHEST) / (_INC - 1)
    dvar_c = jnp.sum(jnp.where(eye_b, cov, 0.0), axis=1, keepdims=True)
    dvar_r = jnp.sum(jnp.where(eye_b, cov, 0.0), axis=0, keepdims=True)
    adj = cov / jnp.sqrt(dvar_c) / jnp.sqrt(dvar_r)
    A = (adj > 0.5).astype(f32)

    ones_c = jnp.ones((_N, 1), f32)

    cnt = _dot_m(A, ones_c)
    xWl = _dot_t(x, c1_Wl_ref[...])
    mean1 = _dot_m(A, xWl) / jnp.maximum(cnt, 1.0)
    h = jax.nn.relu(mean1 + c1_bl_ref[...] + _dot_t(x, c1_Wr_ref[...]))

    score1 = (_dot_m(A, _dot_t(h, p1_Wrel_ref[...])) + p1_brel_ref[...]
              + _dot_t(h, p1_Wroot_ref[...]))

    s1_r = _dot_m(score1, eye_f)
    beats1 = (s1_r > score1) | ((s1_r == score1) & (col_i < row_i))
    rank1 = jnp.sum(beats1.astype(f32), axis=1, keepdims=True)
    sel1 = (rank1 < _K1).astype(f32)
    sel1_r = _dot_m(sel1, eye_f)

    x2 = h * jnp.tanh(score1) * sel1
    A1 = A * sel1 * sel1_r
    cnt1 = _dot_m(A1, ones_c)

    x2Wl = _dot_t(x2, c2_Wl_ref[...])
    mean2 = _dot_m(A1, x2Wl) / jnp.maximum(cnt1, 1.0)
    h2 = jax.nn.relu(mean2 + c2_bl_ref[...] + _dot_t(x2, c2_Wr_ref[...])) * sel1

    score2 = (_dot_m(A1, _dot_t(h2, p2_Wrel_ref[...])) + p2_brel_ref[...]
              + _dot_t(h2, p2_Wroot_ref[...]))
    s2_r = _dot_m(score2, eye_f)
    valid_r = sel1_r > 0.0
    beats2 = valid_r & ((s2_r > score2) | ((s2_r == score2) & (col_i < row_i)))
    rank2 = jnp.sum(beats2.astype(f32), axis=1, keepdims=True)
    sel2 = ((rank2 < _K2) & (sel1 > 0.0)).astype(f32)

    w2 = jnp.tanh(score2) * sel2
    g_ref[...] = (_dot_m(w2, h2) / float(_K2)).reshape(1, 1, _OUT)


def _head_body(g_ref, fc1_W_ref, fc1_b_ref, bn1_g_ref, bn1_b_ref,
               fc2_W_ref, fc2_b_ref, bn2_g_ref, bn2_b_ref,
               fc3_W_ref, fc3_b_ref, out_ref):
    c = np.float32(np.sqrt(1.0 + 1e-5))
    g = g_ref[...]
    z = jax.nn.relu(_dot_t(g, fc1_W_ref[...]) + fc1_b_ref[...])
    z = z / c * bn1_g_ref[...] + bn1_b_ref[...]
    z = jax.nn.relu(_dot_t(z, fc2_W_ref[...]) + fc2_b_ref[...])
    z = z / c * bn2_g_ref[...] + bn2_b_ref[...]
    z = _dot_t(z, fc3_W_ref[...]) + fc3_b_ref[...]
    m = jnp.max(z, axis=1, keepdims=True)
    e = jnp.exp(z - m)
    out_ref[...] = e / jnp.sum(e, axis=1, keepdims=True)


def _full(shape):
    return pl.BlockSpec(shape, lambda b: tuple(0 for _ in shape))


def kernel(input, c1_Wl, c1_bl, c1_Wr, p1_Wrel, p1_brel, p1_Wroot,
           c2_Wl, c2_bl, c2_Wr, p2_Wrel, p2_brel, p2_Wroot,
           fc1_W, fc1_b, bn1_g, bn1_b, fc2_W, fc2_b, bn2_g, bn2_b,
           fc3_W, fc3_b):
    f32 = jnp.float32
    x = input.astype(f32)

    g = pl.pallas_call(
        _graph_body,
        grid=(_B,),
        in_specs=[
            pl.BlockSpec((1, _N, _INC), lambda b: (b, 0, 0)),
            _full((_HID, _INC)), _full((1, _HID)), _full((_HID, _INC)),
            _full((1, _HID)), _full((1, 1)), _full((1, _HID)),
            _full((_OUT, _HID)), _full((1, _OUT)), _full((_OUT, _HID)),
            _full((1, _OUT)), _full((1, 1)), _full((1, _OUT)),
        ],
        out_specs=pl.BlockSpec((1, 1, _OUT), lambda b: (b, 0, 0)),
        out_shape=jax.ShapeDtypeStruct((_B, 1, _OUT), f32),
        compiler_params=pltpu.CompilerParams(
            dimension_semantics=("parallel",)),
    )(x, c1_Wl, c1_bl.reshape(1, _HID), c1_Wr,
      p1_Wrel, p1_brel.reshape(1, 1), p1_Wroot,
      c2_Wl, c2_bl.reshape(1, _OUT), c2_Wr,
      p2_Wrel, p2_brel.reshape(1, 1), p2_Wroot)

    out = pl.pallas_call(
        _head_body,
        out_shape=jax.ShapeDtypeStruct((_B, 2), f32),
    )(g.reshape(_B, _OUT), fc1_W, fc1_b.reshape(1, 512), bn1_g.reshape(1, 512),
      bn1_b.reshape(1, 512), fc2_W, fc2_b.reshape(1, 256),
      bn2_g.reshape(1, 256), bn2_b.reshape(1, 256),
      fc3_W, fc3_b.reshape(1, 2))
    return out


# speedup vs baseline: 3392.4711x; 1.2751x over previous
import numpy as np
import jax
import jax.numpy as jnp
from jax.experimental import pallas as pl
from jax.experimental.pallas import tpu as pltpu

_B, _N, _INC, _HID, _OUT = 64, 512, 195, 128, 64
_K1, _K2 = 256, 128
_GPP = 2


def _dot_t(a, b, precision=None):
    return jax.lax.dot_general(a, b, (((1,), (1,)), ((), ())),
                               precision=precision,
                               preferred_element_type=jnp.float32)


def _dot_m(a, b, precision=None):
    return jax.lax.dot_general(a, b, (((0,), (0,)), ((), ())),
                               precision=precision,
                               preferred_element_type=jnp.float32)


def _dot_b(a, b):
    return jax.lax.dot_general(a, b, (((1,), (0,)), ((), ())),
                               preferred_element_type=jnp.float32)


def _graph_body(x_ref, c1_Wl_ref, c1_bl_ref, c1_Wr_ref,
                p1_W2_ref, p1_brel_ref,
                c2_Wl_ref, c2_bl_ref, c2_Wr_ref,
                p2_W2_ref, p2_brel_ref,
                g_ref):
    for i in range(_GPP):
        _one_graph(i, x_ref, c1_Wl_ref, c1_bl_ref, c1_Wr_ref,
                   p1_W2_ref, p1_brel_ref,
                   c2_Wl_ref, c2_bl_ref, c2_Wr_ref,
                   p2_W2_ref, p2_brel_ref, g_ref)


def _one_graph(i, x_ref, c1_Wl_ref, c1_bl_ref, c1_Wr_ref,
               p1_W2_ref, p1_brel_ref,
               c2_Wl_ref, c2_bl_ref, c2_Wr_ref,
               p2_W2_ref, p2_brel_ref,
               g_ref):
    f32 = jnp.float32
    x = x_ref[i]

    row_i = jax.lax.broadcasted_iota(jnp.int32, (_N, _N), 0)
    col_i = jax.lax.broadcasted_iota(jnp.int32, (_N, _N), 1)
    tri = col_i < row_i

    xm = x - jnp.mean(x, axis=1, keepdims=True)
    cov = _dot_t(xm, xm)
    d_c = jnp.sqrt(jnp.sum(xm * xm, axis=1, keepdims=True))
    d_r = jnp.transpose(d_c)
    adj_b = cov > (0.5 * d_c) * d_r
    A = adj_b.astype(f32)
    ones_c = jnp.ones((_N, 1), f32)

    cnt = _dot_m(A, ones_c)
    xWl = _dot_t(x, c1_Wl_ref[...])
    mean1 = _dot_m(A, xWl) / jnp.maximum(cnt, 1.0)
    h = jax.nn.relu(mean1 + c1_bl_ref[...] + _dot_t(x, c1_Wr_ref[...]))

    score1 = (_dot_m(A, _dot_t(h, p1_W2_ref[0:1, :])) + p1_brel_ref[...]
              + _dot_t(h, p1_W2_ref[1:2, :]))

    s1_r = jnp.transpose(score1)
    beats1 = (s1_r > score1) | ((s1_r == score1) & tri)
    rank1 = jnp.sum(beats1.astype(f32), axis=1, keepdims=True)
    sel1_b = rank1 < _K1
    sel1 = sel1_b.astype(f32)
    sel1_r = jnp.transpose(sel1)

    x2 = h * jnp.tanh(score1) * sel1
    cnt1 = _dot_m(A, sel1)

    x2Wl = _dot_t(x2, c2_Wl_ref[...])
    mean2 = _dot_m(A, x2Wl) / jnp.maximum(cnt1, 1.0)
    h2 = jax.nn.relu(mean2 + c2_bl_ref[...] + _dot_t(x2, c2_Wr_ref[...])) * sel1

    score2 = (_dot_m(A, _dot_t(h2, p2_W2_ref[0:1, :])) + p2_brel_ref[...]
              + _dot_t(h2, p2_W2_ref[1:2, :]))
    s2_r = jnp.transpose(score2)
    valid_r = sel1_r > 0.0
    beats2 = valid_r & ((s2_r > score2) | ((s2_r == score2) & tri))
    rank2 = jnp.sum(beats2.astype(f32), axis=1, keepdims=True)
    sel2 = ((rank2 < _K2) & sel1_b).astype(f32)

    w2 = jnp.tanh(score2) * sel2
    g_ref[i] = _dot_m(w2, h2) / float(_K2)


def _head_body(g_ref, fc1_W_ref, fc1_b_ref, bn1_g_ref, bn1_b_ref,
               fc2_W_ref, fc2_b_ref, bn2_g_ref, bn2_b_ref,
               fc3_W_ref, fc3_b_ref, out_ref):
    c = np.float32(np.sqrt(1.0 + 1e-5))
    g = g_ref[...]
    z = jax.nn.relu(_dot_t(g, fc1_W_ref[...]) + fc1_b_ref[...])
    z = z / c * bn1_g_ref[...] + bn1_b_ref[...]
    z = jax.nn.relu(_dot_t(z, fc2_W_ref[...]) + fc2_b_ref[...])
    z = z / c * bn2_g_ref[...] + bn2_b_ref[...]
    z = _dot_t(z, fc3_W_ref[...]) + fc3_b_ref[...]
    m = jnp.max(z, axis=1, keepdims=True)
    e = jnp.exp(z - m)
    out_ref[...] = e / jnp.sum(e, axis=1, keepdims=True)


def _full(shape):
    return pl.BlockSpec(shape, lambda b: tuple(0 for _ in shape))


def kernel(input, c1_Wl, c1_bl, c1_Wr, p1_Wrel, p1_brel, p1_Wroot,
           c2_Wl, c2_bl, c2_Wr, p2_Wrel, p2_brel, p2_Wroot,
           fc1_W, fc1_b, bn1_g, bn1_b, fc2_W, fc2_b, bn2_g, bn2_b,
           fc3_W, fc3_b):
    f32 = jnp.float32
    x = input.astype(f32)

    g = pl.pallas_call(
        _graph_body,
        grid=(_B // _GPP,),
        in_specs=[
            pl.BlockSpec((_GPP, _N, _INC), lambda b: (b, 0, 0)),
            _full((_HID, _INC)), _full((1, _HID)), _full((_HID, _INC)),
            _full((2, _HID)), _full((1, 1)),
            _full((_OUT, _HID)), _full((1, _OUT)), _full((_OUT, _HID)),
            _full((2, _OUT)), _full((1, 1)),
        ],
        out_specs=pl.BlockSpec((_GPP, 1, _OUT), lambda b: (b, 0, 0)),
        out_shape=jax.ShapeDtypeStruct((_B, 1, _OUT), f32),
        compiler_params=pltpu.CompilerParams(
            dimension_semantics=("parallel",)),
    )(x, c1_Wl, c1_bl.reshape(1, _HID), c1_Wr,
      jnp.concatenate([p1_Wrel, p1_Wroot], axis=0), p1_brel.reshape(1, 1),
      c2_Wl, c2_bl.reshape(1, _OUT), c2_Wr,
      jnp.concatenate([p2_Wrel, p2_Wroot], axis=0), p2_brel.reshape(1, 1))

    out = pl.pallas_call(
        _head_body,
        out_shape=jax.ShapeDtypeStruct((_B, 2), f32),
    )(g.reshape(_B, _OUT), fc1_W, fc1_b.reshape(1, 512), bn1_g.reshape(1, 512),
      bn1_b.reshape(1, 512), fc2_W, fc2_b.reshape(1, 256),
      bn2_g.reshape(1, 256), bn2_b.reshape(1, 256),
      fc3_W, fc3_b.reshape(1, 2))
    return out
